# baseline (speedup 1.0000x reference)
import functools

import jax
import jax.numpy as jnp
from jax.experimental import pallas as pl
from jax.experimental.pallas import tpu as pltpu

_VOCAB = 4096
_R = 8


def _body(idx_ref, tgt_ref, *refs):
    table_refs = refs[:_R]
    out_ref = refs[_R]
    loss_ref = refs[_R + 1]
    i = pl.program_id(0)

    rows = jnp.concatenate([r[0] for r in table_refs], axis=0)
    out_ref[...] = rows

    mx = jnp.max(rows, axis=1, keepdims=True)
    se = jnp.sum(jnp.exp(rows - mx), axis=1, keepdims=True)
    lse = mx + jnp.log(se)

    lane = jax.lax.broadcasted_iota(jnp.int32, (1, _VOCAB), 1)
    total = jnp.float32(0.0)
    for k in range(_R):
        t = tgt_ref[i * _R + k]
        pick = jnp.sum(jnp.where(lane == t, rows[k : k + 1, :], 0.0))
        total += lse[k, 0] - pick

    @pl.when(i == 0)
    def _init():
        loss_ref[...] = jnp.zeros_like(loss_ref)

    loss_ref[...] += total.reshape(1, 1)

    @pl.when(i == pl.num_programs(0) - 1)
    def _fin():
        loss_ref[...] = loss_ref[...] / (pl.num_programs(0) * _R)


@jax.jit
def kernel(index, target, table):
    b, s = index.shape
    n = b * s
    v = table.shape[1]
    flat_idx = index.reshape(n)
    flat_tgt = target.reshape(n)
    steps = n // _R

    grid_spec = pltpu.PrefetchScalarGridSpec(
        num_scalar_prefetch=2,
        grid=(steps,),
        in_specs=[
            pl.BlockSpec(
                (1, 1, v),
                functools.partial(
                    lambda k, i, idx_ref, tgt_ref: (idx_ref[i * _R + k], 0, 0), k
                ),
            )
            for k in range(_R)
        ],
        out_specs=[
            pl.BlockSpec((_R, v), lambda i, idx_ref, tgt_ref: (i, 0)),
            pl.BlockSpec((1, 1), lambda i, idx_ref, tgt_ref: (0, 0)),
        ],
    )

    logits_flat, loss = pl.pallas_call(
        _body,
        grid_spec=grid_spec,
        out_shape=[
            jax.ShapeDtypeStruct((n, v), jnp.float32),
            jax.ShapeDtypeStruct((1, 1), jnp.float32),
        ],
    )(flat_idx, flat_tgt, *([table.reshape(table.shape[0], 1, v)] * _R))

    return logits_flat.reshape(b, s, v), loss[0, 0]


# speedup vs baseline: 1.5595x; 1.5595x over previous
import functools

import jax
import jax.numpy as jnp
from jax.experimental import pallas as pl
from jax.experimental.pallas import tpu as pltpu

_VOCAB = 4096
_R = 16


def _body(idx_ref, tgt_ref, *refs):
    table_refs = refs[:_R]
    out_ref = refs[_R]
    loss_ref = refs[_R + 1]
    i = pl.program_id(0)

    rows = jnp.concatenate([r[0] for r in table_refs], axis=0)
    out_ref[...] = rows

    mx = jnp.max(rows, axis=1, keepdims=True)
    se = jnp.sum(jnp.exp(rows - mx), axis=1, keepdims=True)
    lse = mx + jnp.log(se)

    lane = jax.lax.broadcasted_iota(jnp.int32, (_R, _VOCAB), 1)
    tgts = jnp.concatenate(
        [tgt_ref[i * _R + k].reshape(1, 1) for k in range(_R)], axis=0
    )
    pick = jnp.sum(
        jnp.where(lane == tgts, rows, 0.0), axis=1, keepdims=True
    )
    total = jnp.sum(lse - pick)

    @pl.when(i == 0)
    def _init():
        loss_ref[...] = jnp.zeros_like(loss_ref)

    loss_ref[...] += total.reshape(1, 1)

    @pl.when(i == pl.num_programs(0) - 1)
    def _fin():
        loss_ref[...] = loss_ref[...] / (pl.num_programs(0) * _R)


@jax.jit
def kernel(index, target, table):
    b, s = index.shape
    n = b * s
    v = table.shape[1]
    flat_idx = index.reshape(n)
    flat_tgt = target.reshape(n)
    steps = n // _R

    grid_spec = pltpu.PrefetchScalarGridSpec(
        num_scalar_prefetch=2,
        grid=(steps,),
        in_specs=[
            pl.BlockSpec(
                (1, 1, v),
                functools.partial(
                    lambda k, i, idx_ref, tgt_ref: (idx_ref[i * _R + k], 0, 0), k
                ),
            )
            for k in range(_R)
        ],
        out_specs=[
            pl.BlockSpec((_R, v), lambda i, idx_ref, tgt_ref: (i, 0)),
            pl.BlockSpec((1, 1), lambda i, idx_ref, tgt_ref: (0, 0)),
        ],
    )

    logits_flat, loss = pl.pallas_call(
        _body,
        grid_spec=grid_spec,
        out_shape=[
            jax.ShapeDtypeStruct((n, v), jnp.float32),
            jax.ShapeDtypeStruct((1, 1), jnp.float32),
        ],
    )(flat_idx, flat_tgt, *([table.reshape(table.shape[0], 1, v)] * _R))

    return logits_flat.reshape(b, s, v), loss[0, 0]


# speedup vs baseline: 3.7486x; 2.4037x over previous
import functools

import jax
import jax.numpy as jnp
from jax import lax
from jax.experimental import pallas as pl
from jax.experimental.pallas import tpu as pltpu
from jax.experimental.pallas import tpu_sc as plsc

_VOCAB = 4096
_N = 8192
_NC = 2
_NS = 16
_NW = _NC * _NS
_BPW = _N // _NW
_C = 8
_NCHUNK = _BPW // _C
_L = 16

_RB = 256
_NBLK = _N // _RB


def _sc_gather_body(
    table_ref,
    tflat_ref,
    idx_ref,
    tgt_ref,
    out_ref,
    pick_ref,
    idx_v,
    tgt_v,
    pidx_v,
    picked_v,
    rows0,
    rows1,
    gsem0,
    gsem1,
    osem0,
    osem1,
    psem,
):
    c = lax.axis_index("c")
    s = lax.axis_index("s")
    wid = s * _NC + c
    base = wid * _BPW

    pltpu.sync_copy(idx_ref.at[pl.ds(base, _BPW)], idx_v)
    pltpu.sync_copy(tgt_ref.at[pl.ds(base, _BPW)], tgt_v)
    for j in range(_BPW // _L):
        sl = pl.ds(j * _L, _L)
        pidx_v[sl] = idx_v[sl] * _VOCAB + tgt_v[sl]

    pick_cps = []
    for j in range(_BPW // 128):
        sl = pl.ds(j * 128, 128)
        pick_cps.append(
            pltpu.async_copy(tflat_ref.at[pidx_v.at[sl]], picked_v.at[sl], psem)
        )

    rows = (rows0, rows1)
    gsem = (gsem0, gsem1)
    osem = (osem0, osem1)
    g_cp = [None, None]
    o_cp = [None, None]

    def start_gather(ci):
        b = ci & 1
        g_cp[b] = pltpu.async_copy(
            table_ref.at[idx_v.at[pl.ds(ci * _C, _C)]], rows[b], gsem[b]
        )

    start_gather(0)
    for ci in range(_NCHUNK):
        b = ci & 1
        nb = 1 - b
        if ci + 1 < _NCHUNK:
            if o_cp[nb] is not None:
                o_cp[nb].wait()
            start_gather(ci + 1)
        g_cp[b].wait()
        o_cp[b] = pltpu.async_copy(
            rows[b], out_ref.at[pl.ds(base + ci * _C, _C)], osem[b]
        )
    for cp in o_cp:
        cp.wait()
    for cp in pick_cps:
        cp.wait()
    pltpu.sync_copy(picked_v, pick_ref.at[pl.ds(base, _BPW)])


def _loss_body(logits_ref, pick_ref, loss_ref):
    i = pl.program_id(0)
    blk = logits_ref[...]
    mx = jnp.max(blk, axis=1, keepdims=True)
    se = jnp.sum(jnp.exp(blk - mx), axis=1, keepdims=True)
    lse = mx + jnp.log(se)
    total = jnp.sum(lse) - jnp.sum(pick_ref[...])

    @pl.when(i == 0)
    def _init():
        loss_ref[...] = jnp.zeros_like(loss_ref)

    loss_ref[...] += total.reshape(1, 1)

    @pl.when(i == pl.num_programs(0) - 1)
    def _fin():
        loss_ref[...] = loss_ref[...] / _N


@jax.jit
def kernel(index, target, table):
    b, s = index.shape
    v = table.shape[1]
    flat_idx = index.reshape(_N)
    flat_tgt = target.reshape(_N)

    mesh = plsc.VectorSubcoreMesh(core_axis_name="c", subcore_axis_name="s")
    sc_gather = pl.kernel(
        _sc_gather_body,
        out_type=(
            jax.ShapeDtypeStruct((_N, v), jnp.float32),
            jax.ShapeDtypeStruct((_N,), jnp.float32),
        ),
        mesh=mesh,
        scratch_types=[
            pltpu.VMEM((_BPW,), jnp.int32),
            pltpu.VMEM((_BPW,), jnp.int32),
            pltpu.VMEM((_BPW,), jnp.int32),
            pltpu.VMEM((_BPW,), jnp.float32),
            pltpu.VMEM((_C, _VOCAB), jnp.float32),
            pltpu.VMEM((_C, _VOCAB), jnp.float32),
            pltpu.SemaphoreType.DMA,
            pltpu.SemaphoreType.DMA,
            pltpu.SemaphoreType.DMA,
            pltpu.SemaphoreType.DMA,
            pltpu.SemaphoreType.DMA,
        ],
    )
    logits_flat, picked = sc_gather(
        table, table.reshape(-1), flat_idx, flat_tgt
    )

    loss = pl.pallas_call(
        _loss_body,
        grid=(_NBLK,),
        in_specs=[
            pl.BlockSpec((_RB, v), lambda i: (i, 0)),
            pl.BlockSpec((1, 1, _RB), lambda i: (i, 0, 0)),
        ],
        out_specs=pl.BlockSpec((1, 1), lambda i: (0, 0)),
        out_shape=jax.ShapeDtypeStruct((1, 1), jnp.float32),
    )(logits_flat, picked.reshape(_NBLK, 1, _RB))

    return logits_flat.reshape(b, s, v), loss[0, 0]


# speedup vs baseline: 4.6356x; 1.2366x over previous
import jax
import jax.numpy as jnp
from jax import lax
from jax.experimental import pallas as pl
from jax.experimental.pallas import tpu as pltpu
from jax.experimental.pallas import tpu_sc as plsc

_VOCAB = 4096
_N = 8192
_NC = 2
_NS = 16
_NW = _NC * _NS
_BPW = _N // _NW
_C = 8
_NCHUNK = _BPW // _C

_RB = 256
_NBLK = _N // _RB


def _sc_gather_body(
    table_ref, idx_ref, out_ref, idx_v, rows0, rows1, gsem0, gsem1, osem0, osem1
):
    c = lax.axis_index("c")
    s = lax.axis_index("s")
    wid = s * _NC + c
    base = wid * _BPW

    pltpu.sync_copy(idx_ref.at[pl.ds(base, _BPW)], idx_v)

    rows = (rows0, rows1)
    gsem = (gsem0, gsem1)
    osem = (osem0, osem1)
    g_cp = [None, None]
    o_cp = [None, None]

    def start_gather(ci):
        b = ci & 1
        g_cp[b] = pltpu.async_copy(
            table_ref.at[idx_v.at[pl.ds(ci * _C, _C)]], rows[b], gsem[b]
        )

    start_gather(0)
    for ci in range(_NCHUNK):
        b = ci & 1
        nb = 1 - b
        if ci + 1 < _NCHUNK:
            if o_cp[nb] is not None:
                o_cp[nb].wait()
            start_gather(ci + 1)
        g_cp[b].wait()
        o_cp[b] = pltpu.async_copy(
            rows[b], out_ref.at[pl.ds(base + ci * _C, _C)], osem[b]
        )
    for cp in o_cp:
        cp.wait()


def _loss_body(logits_ref, tgt_ref, loss_ref):
    i = pl.program_id(0)
    blk = logits_ref[...]
    mx = jnp.max(blk, axis=1, keepdims=True)
    se = jnp.sum(jnp.exp(blk - mx), axis=1, keepdims=True)
    lse = mx + jnp.log(se)
    lane = lax.broadcasted_iota(jnp.int32, (_RB, _VOCAB), 1)
    pick = jnp.sum(
        jnp.where(lane == tgt_ref[...], blk, 0.0), axis=1, keepdims=True
    )
    total = jnp.sum(lse - pick)

    @pl.when(i == 0)
    def _init():
        loss_ref[...] = jnp.zeros_like(loss_ref)

    loss_ref[...] += total.reshape(1, 1)

    @pl.when(i == pl.num_programs(0) - 1)
    def _fin():
        loss_ref[...] = loss_ref[...] / _N


@jax.jit
def kernel(index, target, table):
    b, s = index.shape
    v = table.shape[1]
    flat_idx = index.reshape(_N)

    mesh = plsc.VectorSubcoreMesh(core_axis_name="c", subcore_axis_name="s")
    sc_gather = pl.kernel(
        _sc_gather_body,
        out_type=jax.ShapeDtypeStruct((_N, v), jnp.float32),
        mesh=mesh,
        scratch_types=[
            pltpu.VMEM((_BPW,), jnp.int32),
            pltpu.VMEM((_C, _VOCAB), jnp.float32),
            pltpu.VMEM((_C, _VOCAB), jnp.float32),
            pltpu.SemaphoreType.DMA,
            pltpu.SemaphoreType.DMA,
            pltpu.SemaphoreType.DMA,
            pltpu.SemaphoreType.DMA,
        ],
    )
    logits_flat = sc_gather(table, flat_idx)

    loss = pl.pallas_call(
        _loss_body,
        grid=(_NBLK,),
        in_specs=[
            pl.BlockSpec((_RB, v), lambda i: (i, 0)),
            pl.BlockSpec((_RB, 1), lambda i: (i, 0)),
        ],
        out_specs=pl.BlockSpec((1, 1), lambda i: (0, 0)),
        out_shape=jax.ShapeDtypeStruct((1, 1), jnp.float32),
    )(logits_flat, target.reshape(_N, 1))

    return logits_flat.reshape(b, s, v), loss[0, 0]


# speedup vs baseline: 4.9953x; 1.0776x over previous
import jax
import jax.numpy as jnp
from jax import lax
from jax.experimental import pallas as pl
from jax.experimental.pallas import tpu as pltpu
from jax.experimental.pallas import tpu_sc as plsc

_VOCAB = 4096
_N = 8192
_NC = 2
_NS = 16
_NW = _NC * _NS
_BPW = _N // _NW
_C = 8
_NB = 3
_NCHUNK = _BPW // _C

_RB = 512
_NBLK = _N // _RB


def _sc_gather_body(
    table_ref,
    idx_ref,
    tgt_ref,
    out_ref,
    pick_ref,
    idx_v,
    tgt_v,
    pick_v,
    rows0,
    rows1,
    rows2,
    gsem0,
    gsem1,
    gsem2,
    osem0,
    osem1,
    osem2,
):
    c = lax.axis_index("c")
    s = lax.axis_index("s")
    wid = s * _NC + c
    base = wid * _BPW

    pltpu.sync_copy(idx_ref.at[pl.ds(base, _BPW)], idx_v)
    pltpu.sync_copy(tgt_ref.at[pl.ds(base, _BPW)], tgt_v.at[pl.ds(0, _BPW)])

    rows = (rows0, rows1, rows2)
    gsem = (gsem0, gsem1, gsem2)
    osem = (osem0, osem1, osem2)
    g_cp = [None] * _NB
    o_cp = [None] * _NB

    lanes = lax.broadcasted_iota(jnp.int32, (16,), 0)
    row_idx = lanes & (_C - 1)
    lo_mask = lanes < _C

    def start_gather(k):
        b = k % _NB
        g_cp[b] = pltpu.async_copy(
            table_ref.at[idx_v.at[pl.ds(k * _C, _C)]], rows[b], gsem[b]
        )

    for k in range(_NB):
        start_gather(k)

    for j in range(_NCHUNK):
        b = j % _NB
        g_cp[b].wait()
        col = tgt_v[pl.ds(j * _C, 16)] & (_VOCAB - 1)
        g = plsc.load_gather(rows[b], [row_idx, col], mask=lo_mask)
        plsc.store_compressed(pick_v.at[pl.ds(j * _C, 16)], g, mask=lo_mask)
        o_cp[b] = pltpu.async_copy(
            rows[b], out_ref.at[pl.ds(base + j * _C, _C)], osem[b]
        )
        k = j + _NB
        if k < _NCHUNK:
            o_cp[b].wait()
            start_gather(k)
    for cp in o_cp:
        cp.wait()
    pltpu.sync_copy(pick_v.at[pl.ds(0, _BPW)], pick_ref.at[pl.ds(base, _BPW)])


def _loss_body(logits_ref, pick_ref, ones_ref, loss_ref):
    i = pl.program_id(0)
    blk = logits_ref[...]
    mx = jnp.max(blk, axis=1, keepdims=True)
    e = jnp.exp(blk - mx)
    se = jax.lax.dot_general(
        e, ones_ref[...], (((1,), (0,)), ((), ())),
        preferred_element_type=jnp.float32,
    )
    lse = mx + jnp.log(se[:, :1])
    total = jnp.sum(lse) - jnp.sum(pick_ref[...])

    @pl.when(i == 0)
    def _init():
        loss_ref[...] = jnp.zeros_like(loss_ref)

    loss_ref[...] += total.reshape(1, 1)

    @pl.when(i == pl.num_programs(0) - 1)
    def _fin():
        loss_ref[...] = loss_ref[...] / _N


@jax.jit
def kernel(index, target, table):
    b, s = index.shape
    v = table.shape[1]
    flat_idx = index.reshape(_N)
    flat_tgt = target.reshape(_N)

    mesh = plsc.VectorSubcoreMesh(core_axis_name="c", subcore_axis_name="s")
    sc_gather = pl.kernel(
        _sc_gather_body,
        out_type=(
            jax.ShapeDtypeStruct((_N, v), jnp.float32),
            jax.ShapeDtypeStruct((_N,), jnp.float32),
        ),
        mesh=mesh,
        compiler_params=pltpu.CompilerParams(needs_layout_passes=False),
        scratch_types=[
            pltpu.VMEM((_BPW,), jnp.int32),
            pltpu.VMEM((_BPW + 16, ), jnp.int32),
            pltpu.VMEM((_BPW + 16,), jnp.float32),
            pltpu.VMEM((_C, _VOCAB), jnp.float32),
            pltpu.VMEM((_C, _VOCAB), jnp.float32),
            pltpu.VMEM((_C, _VOCAB), jnp.float32),
            pltpu.SemaphoreType.DMA,
            pltpu.SemaphoreType.DMA,
            pltpu.SemaphoreType.DMA,
            pltpu.SemaphoreType.DMA,
            pltpu.SemaphoreType.DMA,
            pltpu.SemaphoreType.DMA,
        ],
    )
    logits_flat, picked = sc_gather(table, flat_idx, flat_tgt)

    loss = pl.pallas_call(
        _loss_body,
        grid=(_NBLK,),
        in_specs=[
            pl.BlockSpec((_RB, v), lambda i: (i, 0)),
            pl.BlockSpec((1, 1, _RB), lambda i: (i, 0, 0)),
            pl.BlockSpec((v, 128), lambda i: (0, 0)),
        ],
        out_specs=pl.BlockSpec((1, 1), lambda i: (0, 0)),
        out_shape=jax.ShapeDtypeStruct((1, 1), jnp.float32),
    )(
        logits_flat,
        picked.reshape(_NBLK, 1, _RB),
        jnp.ones((v, 128), jnp.float32),
    )

    return logits_flat.reshape(b, s, v), loss[0, 0]


# speedup vs baseline: 5.5685x; 1.1148x over previous
import jax
import jax.numpy as jnp
from jax import lax
from jax.experimental import pallas as pl
from jax.experimental.pallas import tpu as pltpu
from jax.experimental.pallas import tpu_sc as plsc

_VOCAB = 4096
_N = 8192
_NC = 2
_NS = 16
_NW = _NC * _NS
_BPW = _N // _NW
_C = 8
_NB = 3
_NCHUNK = _BPW // _C
_L = 16

_TRB = 512


def _lse_body(table_ref, lse_ref):
    blk = table_ref[...]
    mx = jnp.max(blk, axis=1, keepdims=True)
    se = jnp.sum(jnp.exp(blk - mx), axis=1, keepdims=True)
    lse_ref[...] = mx + jnp.log(se)


def _sc_gather_body(
    table_ref,
    idx_ref,
    tgt_ref,
    lse_ref,
    out_ref,
    part_ref,
    idx_v,
    tgt_v,
    lse_v,
    part_v,
    rows0,
    rows1,
    rows2,
    gsem0,
    gsem1,
    gsem2,
    osem0,
    osem1,
    osem2,
):
    c = lax.axis_index("c")
    s = lax.axis_index("s")
    wid = s * _NC + c
    base = wid * _BPW

    pltpu.sync_copy(idx_ref.at[pl.ds(base, _BPW)], idx_v)
    pltpu.sync_copy(tgt_ref.at[pl.ds(base, _BPW)], tgt_v.at[pl.ds(0, _BPW)])
    pltpu.sync_copy(lse_ref, lse_v)

    acc_lse = jnp.zeros((_L,), jnp.float32)
    for j in range(_BPW // _L):
        iv = idx_v[pl.ds(j * _L, _L)]
        acc_lse = acc_lse + plsc.load_gather(lse_v, [iv])

    rows = (rows0, rows1, rows2)
    gsem = (gsem0, gsem1, gsem2)
    osem = (osem0, osem1, osem2)
    g_cp = [None] * _NB
    o_cp = [None] * _NB

    lanes = lax.broadcasted_iota(jnp.int32, (_L,), 0)
    row_idx = lanes & (_C - 1)
    lo_mask = lanes < _C
    zero = jnp.zeros((_L,), jnp.float32)
    acc_pick = zero

    def start_gather(k):
        b = k % _NB
        g_cp[b] = pltpu.async_copy(
            table_ref.at[idx_v.at[pl.ds(k * _C, _C)]], rows[b], gsem[b]
        )

    for k in range(_NB):
        start_gather(k)

    for j in range(_NCHUNK):
        b = j % _NB
        g_cp[b].wait()
        col = tgt_v[pl.ds(j * _C, _L)] & (_VOCAB - 1)
        g = plsc.load_gather(rows[b], [row_idx, col], mask=lo_mask)
        acc_pick = acc_pick + jnp.where(lo_mask, g, zero)
        o_cp[b] = pltpu.async_copy(
            rows[b], out_ref.at[pl.ds(base + j * _C, _C)], osem[b]
        )
        k = j + _NB
        if k < _NCHUNK:
            o_cp[b].wait()
            start_gather(k)
    for cp in o_cp:
        cp.wait()
    part_v[...] = acc_lse - acc_pick
    pltpu.sync_copy(part_v, part_ref.at[pl.ds(wid * _L, _L)])


@jax.jit
def kernel(index, target, table):
    b, s = index.shape
    v = table.shape[1]
    flat_idx = index.reshape(_N)
    flat_tgt = target.reshape(_N)

    lse = pl.pallas_call(
        _lse_body,
        grid=(_VOCAB // _TRB,),
        in_specs=[pl.BlockSpec((_TRB, v), lambda i: (i, 0))],
        out_specs=pl.BlockSpec((_TRB, 1), lambda i: (i, 0)),
        out_shape=jax.ShapeDtypeStruct((_VOCAB, 1), jnp.float32),
    )(table)

    mesh = plsc.VectorSubcoreMesh(core_axis_name="c", subcore_axis_name="s")
    sc_gather = pl.kernel(
        _sc_gather_body,
        out_type=(
            jax.ShapeDtypeStruct((_N, v), jnp.float32),
            jax.ShapeDtypeStruct((_NW * _L,), jnp.float32),
        ),
        mesh=mesh,
        compiler_params=pltpu.CompilerParams(needs_layout_passes=False),
        scratch_types=[
            pltpu.VMEM((_BPW,), jnp.int32),
            pltpu.VMEM((_BPW + _L,), jnp.int32),
            pltpu.VMEM((_VOCAB,), jnp.float32),
            pltpu.VMEM((_L,), jnp.float32),
            pltpu.VMEM((_C, _VOCAB), jnp.float32),
            pltpu.VMEM((_C, _VOCAB), jnp.float32),
            pltpu.VMEM((_C, _VOCAB), jnp.float32),
            pltpu.SemaphoreType.DMA,
            pltpu.SemaphoreType.DMA,
            pltpu.SemaphoreType.DMA,
            pltpu.SemaphoreType.DMA,
            pltpu.SemaphoreType.DMA,
            pltpu.SemaphoreType.DMA,
        ],
    )
    logits_flat, partials = sc_gather(table, flat_idx, flat_tgt, lse.reshape(_VOCAB))

    return logits_flat.reshape(b, s, v), jnp.sum(partials) / _N
